# initial kernel scaffold (unmeasured)
import jax
import jax.numpy as jnp
from jax import lax
from jax.experimental import pallas as pl
from jax.experimental.pallas import tpu as pltpu


def kernel(
    x,
):
    def body(*refs):
        pass

    out_shape = jax.ShapeDtypeStruct(..., jnp.float32)
    return pl.pallas_call(body, out_shape=out_shape)(...)


# baseline (device time: 8735 ns/iter reference)
import jax
import jax.numpy as jnp
from jax import lax
from jax.experimental import pallas as pl
from jax.experimental.pallas import tpu as pltpu

N_DEV = 4


def kernel(x):
    m_per, n = x.shape

    def body(x_ref, out_ref, send_buf, recv_buf, send_sems, recv_sems):
        my_pos = lax.axis_index("i")

        t = jnp.sum(x_ref[:, :], axis=0)
        send_buf[:, :] = jnp.broadcast_to(t[None, :], (8, n))

        def send_desc(k, j):
            return pltpu.make_async_remote_copy(
                src_ref=send_buf,
                dst_ref=recv_buf.at[k],
                send_sem=send_sems.at[j - 1],
                recv_sem=recv_sems.at[k],
                device_id=(j,),
                device_id_type=pl.DeviceIdType.MESH,
            )

        for k in range(N_DEV - 1):
            for j in range(k + 1, N_DEV):
                @pl.when(my_pos == k)
                def _(k=k, j=j):
                    send_desc(k, j).start()

        y = x_ref[:, :]
        s = 1
        while s < m_per:
            y = y + jnp.concatenate(
                [jnp.zeros((s, n), y.dtype), y[: m_per - s, :]], axis=0
            )
            s *= 2

        for k in range(N_DEV - 1):
            @pl.when(k < my_pos)
            def _(k=k):
                recv = pltpu.make_async_remote_copy(
                    src_ref=send_buf,
                    dst_ref=recv_buf.at[k],
                    send_sem=send_sems.at[0],
                    recv_sem=recv_sems.at[k],
                    device_id=(0,),
                    device_id_type=pl.DeviceIdType.MESH,
                )
                recv.wait_recv()

        prefix = jnp.zeros((n,), jnp.float32)
        for k in range(N_DEV - 1):
            prefix = prefix + jnp.where(
                k < my_pos, recv_buf[k, 0, :], jnp.zeros((n,), jnp.float32)
            )

        out_ref[:, :] = y + prefix[None, :]

        for k in range(N_DEV - 1):
            for j in range(k + 1, N_DEV):
                @pl.when(my_pos == k)
                def _(k=k, j=j):
                    send_desc(k, j).wait_send()

    return pl.pallas_call(
        body,
        out_shape=jax.ShapeDtypeStruct((m_per, n), jnp.float32),
        in_specs=[pl.BlockSpec(memory_space=pltpu.VMEM)],
        out_specs=pl.BlockSpec(memory_space=pltpu.VMEM),
        scratch_shapes=[
            pltpu.VMEM((8, n), jnp.float32),
            pltpu.VMEM((N_DEV - 1, 8, n), jnp.float32),
            pltpu.SemaphoreType.DMA((N_DEV - 1,)),
            pltpu.SemaphoreType.DMA((N_DEV - 1,)),
        ],
    )(x)


# device time: 6987 ns/iter; 1.2502x vs baseline; 1.2502x over previous
import jax
import jax.numpy as jnp
from jax import lax
from jax.experimental import pallas as pl
from jax.experimental.pallas import tpu as pltpu

N_DEV = 4


def kernel(x):
    m_per, n = x.shape

    def body(x_ref, out_ref, send_buf, recv_buf, send_sems, recv_sems):
        my_pos = lax.axis_index("i")

        barrier_sem = pltpu.get_barrier_semaphore()
        for k in range(N_DEV - 1):
            for j in range(k + 1, N_DEV):
                @pl.when(my_pos == j)
                def _(k=k):
                    pl.semaphore_signal(
                        barrier_sem, inc=1,
                        device_id=(k,),
                        device_id_type=pl.DeviceIdType.MESH,
                    )
        for k in range(N_DEV - 1):
            @pl.when(my_pos == k)
            def _(k=k):
                pl.semaphore_wait(barrier_sem, N_DEV - 1 - k)

        t = jnp.sum(x_ref[:, :], axis=0)
        send_buf[:, :] = jnp.broadcast_to(t[None, :], (8, n))

        def send_desc(k, j):
            return pltpu.make_async_remote_copy(
                src_ref=send_buf,
                dst_ref=recv_buf.at[k],
                send_sem=send_sems.at[j - 1],
                recv_sem=recv_sems.at[k],
                device_id=(j,),
                device_id_type=pl.DeviceIdType.MESH,
            )

        for k in range(N_DEV - 1):
            for j in range(k + 1, N_DEV):
                @pl.when(my_pos == k)
                def _(k=k, j=j):
                    send_desc(k, j).start()

        y = x_ref[:, :]
        s = 1
        while s < m_per:
            y = y + jnp.concatenate(
                [jnp.zeros((s, n), y.dtype), y[: m_per - s, :]], axis=0
            )
            s *= 2

        for k in range(N_DEV - 1):
            @pl.when(k < my_pos)
            def _(k=k):
                recv = pltpu.make_async_remote_copy(
                    src_ref=send_buf,
                    dst_ref=recv_buf.at[k],
                    send_sem=send_sems.at[0],
                    recv_sem=recv_sems.at[k],
                    device_id=(0,),
                    device_id_type=pl.DeviceIdType.MESH,
                )
                recv.wait_recv()

        prefix = jnp.zeros((n,), jnp.float32)
        for k in range(N_DEV - 1):
            prefix = prefix + jnp.where(
                k < my_pos, recv_buf[k, 0, :], jnp.zeros((n,), jnp.float32)
            )

        out_ref[:, :] = y + prefix[None, :]

        for k in range(N_DEV - 1):
            for j in range(k + 1, N_DEV):
                @pl.when(my_pos == k)
                def _(k=k, j=j):
                    send_desc(k, j).wait_send()

    return pl.pallas_call(
        body,
        out_shape=jax.ShapeDtypeStruct((m_per, n), jnp.float32),
        in_specs=[pl.BlockSpec(memory_space=pltpu.VMEM)],
        out_specs=pl.BlockSpec(memory_space=pltpu.VMEM),
        scratch_shapes=[
            pltpu.VMEM((8, n), jnp.float32),
            pltpu.VMEM((N_DEV - 1, 8, n), jnp.float32),
            pltpu.SemaphoreType.DMA((N_DEV - 1,)),
            pltpu.SemaphoreType.DMA((N_DEV - 1,)),
        ],
        compiler_params=pltpu.CompilerParams(collective_id=0),
    )(x)


# device time: 6978 ns/iter; 1.2518x vs baseline; 1.0013x over previous
import jax
import jax.numpy as jnp
from jax import lax
from jax.experimental import pallas as pl
from jax.experimental.pallas import tpu as pltpu

N_DEV = 4


def kernel(x):
    m_per, n = x.shape

    def body(x_ref, out_ref, send_buf, recv_buf, send_sems, recv_sems):
        my_pos = lax.axis_index("i")

        barrier_sem = pltpu.get_barrier_semaphore()
        for k in range(N_DEV - 1):
            for j in range(k + 1, N_DEV):
                @pl.when(my_pos == j)
                def _(k=k):
                    pl.semaphore_signal(
                        barrier_sem, inc=1,
                        device_id=(k,),
                        device_id_type=pl.DeviceIdType.MESH,
                    )
        for k in range(N_DEV - 1):
            @pl.when(my_pos == k)
            def _(k=k):
                pl.semaphore_wait(barrier_sem, N_DEV - 1 - k)

        t = jnp.sum(x_ref[:, :], axis=0)
        send_buf[:, :] = jnp.broadcast_to(t[None, :], (8, n))

        def send_desc(k, j):
            return pltpu.make_async_remote_copy(
                src_ref=send_buf,
                dst_ref=recv_buf.at[k],
                send_sem=send_sems.at[j - 1],
                recv_sem=recv_sems.at[k],
                device_id=(j,),
                device_id_type=pl.DeviceIdType.MESH,
            )

        for k in range(N_DEV - 1):
            for j in range(k + 1, N_DEV):
                @pl.when(my_pos == k)
                def _(k=k, j=j):
                    send_desc(k, j).start()

        row = lax.broadcasted_iota(jnp.int32, (m_per, m_per), 0)
        col = lax.broadcasted_iota(jnp.int32, (m_per, m_per), 1)
        tri = (row >= col).astype(jnp.float32)
        y = lax.dot_general(
            tri, x_ref[:, :],
            dimension_numbers=(((1,), (0,)), ((), ())),
            preferred_element_type=jnp.float32,
        )

        for k in range(N_DEV - 1):
            @pl.when(k < my_pos)
            def _(k=k):
                recv = pltpu.make_async_remote_copy(
                    src_ref=send_buf,
                    dst_ref=recv_buf.at[k],
                    send_sem=send_sems.at[0],
                    recv_sem=recv_sems.at[k],
                    device_id=(0,),
                    device_id_type=pl.DeviceIdType.MESH,
                )
                recv.wait_recv()

        prefix = jnp.zeros((n,), jnp.float32)
        for k in range(N_DEV - 1):
            prefix = prefix + jnp.where(
                k < my_pos, recv_buf[k, 0, :], jnp.zeros((n,), jnp.float32)
            )

        out_ref[:, :] = y + prefix[None, :]

        for k in range(N_DEV - 1):
            for j in range(k + 1, N_DEV):
                @pl.when(my_pos == k)
                def _(k=k, j=j):
                    send_desc(k, j).wait_send()

    return pl.pallas_call(
        body,
        out_shape=jax.ShapeDtypeStruct((m_per, n), jnp.float32),
        in_specs=[pl.BlockSpec(memory_space=pltpu.VMEM)],
        out_specs=pl.BlockSpec(memory_space=pltpu.VMEM),
        scratch_shapes=[
            pltpu.VMEM((8, n), jnp.float32),
            pltpu.VMEM((N_DEV - 1, 8, n), jnp.float32),
            pltpu.SemaphoreType.DMA((N_DEV - 1,)),
            pltpu.SemaphoreType.DMA((N_DEV - 1,)),
        ],
        compiler_params=pltpu.CompilerParams(collective_id=0),
    )(x)


# device time: 5372 ns/iter; 1.6260x vs baseline; 1.2990x over previous
import jax
import jax.numpy as jnp
from jax import lax
from jax.experimental import pallas as pl
from jax.experimental.pallas import tpu as pltpu

N_DEV = 4


def kernel(x):
    m_per, n = x.shape

    def body(x_hbm, out_ref, x_ref, send_buf, recv_buf, in_sem,
             send_sems, recv_sems):
        my_pos = lax.axis_index("i")

        cp_in = pltpu.make_async_copy(x_hbm, x_ref, in_sem)
        cp_in.start()

        barrier_sem = pltpu.get_barrier_semaphore()
        for k in range(N_DEV - 1):
            for j in range(k + 1, N_DEV):
                @pl.when(my_pos == j)
                def _(k=k):
                    pl.semaphore_signal(
                        barrier_sem, inc=1,
                        device_id=(k,),
                        device_id_type=pl.DeviceIdType.MESH,
                    )

        cp_in.wait()

        @pl.when(my_pos < N_DEV - 1)
        def _():
            send_buf[0, :] = jnp.sum(x_ref[:, :], axis=0)

        for k in range(N_DEV - 1):
            @pl.when(my_pos == k)
            def _(k=k):
                pl.semaphore_wait(barrier_sem, N_DEV - 1 - k)

        def send_desc(k, j):
            return pltpu.make_async_remote_copy(
                src_ref=send_buf,
                dst_ref=recv_buf.at[k],
                send_sem=send_sems.at[j - 1],
                recv_sem=recv_sems.at[k],
                device_id=(j,),
                device_id_type=pl.DeviceIdType.MESH,
            )

        for k in range(N_DEV - 1):
            for j in range(k + 1, N_DEV):
                @pl.when(my_pos == k)
                def _(k=k, j=j):
                    send_desc(k, j).start()

        row = lax.broadcasted_iota(jnp.int32, (m_per, m_per), 0)
        col = lax.broadcasted_iota(jnp.int32, (m_per, m_per), 1)
        tri = (row >= col).astype(jnp.float32)
        y = lax.dot_general(
            tri, x_ref[:, :],
            dimension_numbers=(((1,), (0,)), ((), ())),
            preferred_element_type=jnp.float32,
        )

        for k in range(N_DEV - 1):
            @pl.when(k < my_pos)
            def _(k=k):
                recv = pltpu.make_async_remote_copy(
                    src_ref=send_buf,
                    dst_ref=recv_buf.at[k],
                    send_sem=send_sems.at[0],
                    recv_sem=recv_sems.at[k],
                    device_id=(0,),
                    device_id_type=pl.DeviceIdType.MESH,
                )
                recv.wait_recv()

        prefix = jnp.zeros((n,), jnp.float32)
        for k in range(N_DEV - 1):
            prefix = prefix + jnp.where(
                k < my_pos, recv_buf[k, 0, :], jnp.zeros((n,), jnp.float32)
            )

        out_ref[:, :] = y + prefix[None, :]

        for k in range(N_DEV - 1):
            for j in range(k + 1, N_DEV):
                @pl.when(my_pos == k)
                def _(k=k, j=j):
                    send_desc(k, j).wait_send()

    x = pltpu.with_memory_space_constraint(x, pltpu.MemorySpace.HBM)
    out = pl.pallas_call(
        body,
        out_shape=jax.ShapeDtypeStruct((m_per, n), jnp.float32),
        in_specs=[pl.BlockSpec(memory_space=pltpu.MemorySpace.HBM)],
        out_specs=pl.BlockSpec(memory_space=pltpu.VMEM),
        scratch_shapes=[
            pltpu.VMEM((m_per, n), jnp.float32),
            pltpu.VMEM((1, n), jnp.float32),
            pltpu.VMEM((N_DEV - 1, 1, n), jnp.float32),
            pltpu.SemaphoreType.DMA,
            pltpu.SemaphoreType.DMA((N_DEV - 1,)),
            pltpu.SemaphoreType.DMA((N_DEV - 1,)),
        ],
        compiler_params=pltpu.CompilerParams(collective_id=0),
    )(x)
    return out
